# Initial kernel scaffold; baseline (speedup 1.0000x reference)
import jax
import jax.numpy as jnp
from jax.experimental import pallas as pl


def kernel(x, edge_index, W0a, b0a, W0b, b0b, W1a, b1a, W1b, b1b, W2a, b2a, W2b, b2b, eps):
    raise NotImplementedError("write your pallas kernel here")


# speedup vs baseline: 2.5170x; 2.5170x over previous
import functools

import jax
import jax.numpy as jnp
from jax import lax
from jax.experimental import pallas as pl
from jax.experimental.pallas import tpu as pltpu
from jax.experimental.pallas import tpu_sc as plsc

N = 10000
E = 160000
IN_C = 256
HID = 512
NCLS = 128

NSUB = 16
NCORE = 2
LANE = 128
K = 128
N_PAD = 10240
ROWS_SUB = N_PAD // NSUB
DRAIN_PIECES = ROWS_SUB // 128
EPS_SUB = 10112
CHUNKS = EPS_SUB // K
E_PAD = EPS_SUB * NSUB


def _make_segsum(fc):
    mesh = plsc.VectorSubcoreMesh(core_axis_name="c", subcore_axis_name="s")

    @functools.partial(
        pl.kernel,
        out_type=jax.ShapeDtypeStruct((fc * N_PAD, LANE), jnp.float32),
        mesh=mesh,
        scratch_types=[
            pltpu.VMEM((K,), jnp.int32),
            pltpu.VMEM((K,), jnp.int32),
            pltpu.VMEM((K, LANE), jnp.float32),
            pltpu.VMEM((128, LANE), jnp.float32),
            pltpu.VMEM_SHARED((N_PAD, LANE), jnp.float32),
            pltpu.SemaphoreType.DMA,
        ],
    )
    def segsum(h_hbm, src_hbm, dst_hbm, out_hbm, sidx, didx, rows, zbuf, acc, sem):
        c = lax.axis_index("c")
        s = lax.axis_index("s")
        ebase = s * EPS_SUB

        def zrow(r, carry):
            for k in range(LANE // 16):
                zbuf[r, pl.ds(k * 16, 16)] = jnp.zeros((16,), jnp.float32)
            return carry

        lax.fori_loop(0, 128, zrow, 0)

        def do_pass(p, carry):
            f = p * NCORE + c
            foff = f * N_PAD

            def zpiece(k, carry2):
                pltpu.sync_copy(zbuf, acc.at[pl.ds(s * ROWS_SUB + k * 128, 128)])
                return carry2

            lax.fori_loop(0, DRAIN_PIECES, zpiece, 0)
            plsc.subcore_barrier()

            def chunk(i, carry2):
                base = ebase + i * K
                pltpu.sync_copy(src_hbm.at[pl.ds(base, K)], sidx)
                pltpu.sync_copy(dst_hbm.at[pl.ds(base, K)], didx)
                for j in range(K // 16):
                    sl = pl.ds(j * 16, 16)
                    sidx[sl] = sidx[sl] + foff
                pltpu.async_copy(h_hbm.at[sidx], rows, sem).wait()
                pltpu.sync_copy(rows, acc.at[didx], add=True)
                return carry2

            lax.fori_loop(0, CHUNKS, chunk, 0)
            plsc.subcore_barrier()

            def dpiece(k, carry2):
                r0 = s * ROWS_SUB + k * 128
                pltpu.sync_copy(acc.at[pl.ds(r0, 128)], rows)
                pltpu.sync_copy(rows, out_hbm.at[pl.ds(foff + r0, 128)])
                return carry2

            lax.fori_loop(0, DRAIN_PIECES, dpiece, 0)
            plsc.subcore_barrier()
            return carry

        lax.fori_loop(0, fc // NCORE, do_pass, 0)

    return segsum


_segsum2 = _make_segsum(2)
_segsum4 = _make_segsum(4)


def _make_mlp(fc_in, hid, fc_out, relu_out, bn=512):
    out_c = fc_out * LANE

    def body(scale_ref, agg_ref, h_ref, wa_ref, ba_ref, wb_ref, bb_ref, out_ref):
        scale = scale_ref[0]
        acc = jnp.zeros((bn, hid), jnp.float32)
        for f in range(fc_in):
            t = agg_ref[f] + scale * h_ref[f]
            acc = acc + jnp.dot(t, wa_ref[f], preferred_element_type=jnp.float32)
        h1 = jnp.maximum(acc + ba_ref[...], 0.0)
        h2 = jnp.dot(h1, wb_ref[...], preferred_element_type=jnp.float32) + bb_ref[...]
        if relu_out:
            h2 = jnp.maximum(h2, 0.0)
        for f2 in range(fc_out):
            out_ref[f2] = h2[:, f2 * LANE:(f2 + 1) * LANE]

    return pl.pallas_call(
        body,
        grid=(N_PAD // bn,),
        in_specs=[
            pl.BlockSpec(memory_space=pltpu.SMEM),
            pl.BlockSpec((fc_in, bn, LANE), lambda n: (0, n, 0)),
            pl.BlockSpec((fc_in, bn, LANE), lambda n: (0, n, 0)),
            pl.BlockSpec((fc_in, LANE, hid), lambda n: (0, 0, 0)),
            pl.BlockSpec((1, hid), lambda n: (0, 0)),
            pl.BlockSpec((hid, out_c), lambda n: (0, 0)),
            pl.BlockSpec((1, out_c), lambda n: (0, 0)),
        ],
        out_specs=pl.BlockSpec((fc_out, bn, LANE), lambda n: (0, n, 0)),
        out_shape=jax.ShapeDtypeStruct((fc_out, N_PAD, LANE), jnp.float32),
    )


_mlp0 = _make_mlp(IN_C // LANE, HID, HID // LANE, True)
_mlp1 = _make_mlp(HID // LANE, HID, HID // LANE, True)
_mlp2 = _make_mlp(HID // LANE, NCLS, NCLS // LANE, False)


def kernel(x, edge_index, W0a, b0a, W0b, b0b, W1a, b1a, W1b, b1b, W2a, b2a, W2b, b2b, eps):
    src = edge_index[0]
    dst = edge_index[1]
    pad = E_PAD - E
    srcp = jnp.concatenate([src, jnp.zeros((pad,), jnp.int32)])
    dstp = jnp.concatenate([dst, jnp.full((pad,), N_PAD - 1, jnp.int32)])

    fc0 = IN_C // LANE
    xp = jnp.pad(x, ((0, N_PAD - N), (0, 0)))
    x_flat = xp.reshape(N_PAD, fc0, LANE).transpose(1, 0, 2).reshape(fc0 * N_PAD, LANE)

    scales = (1.0 + eps).reshape(3, 1)

    agg0 = _segsum2(x_flat, srcp, dstp)
    h1 = _mlp0(scales[0], agg0.reshape(fc0, N_PAD, LANE),
               x_flat.reshape(fc0, N_PAD, LANE),
               W0a.reshape(fc0, LANE, HID), b0a.reshape(1, HID),
               W0b, b0b.reshape(1, HID))

    fc1 = HID // LANE
    h1f = h1.reshape(fc1 * N_PAD, LANE)
    agg1 = _segsum4(h1f, srcp, dstp)
    h2 = _mlp1(scales[1], agg1.reshape(fc1, N_PAD, LANE), h1,
               W1a.reshape(fc1, LANE, HID), b1a.reshape(1, HID),
               W1b, b1b.reshape(1, HID))

    h2f = h2.reshape(fc1 * N_PAD, LANE)
    agg2 = _segsum4(h2f, srcp, dstp)
    out = _mlp2(scales[2], agg2.reshape(fc1, N_PAD, LANE), h2,
                W2a.reshape(fc1, LANE, NCLS), b2a.reshape(1, NCLS),
                W2b, b2b.reshape(1, NCLS))

    return out.reshape(N_PAD, NCLS)[:N]


# speedup vs baseline: 2.7767x; 1.1032x over previous
import functools

import jax
import jax.numpy as jnp
from jax import lax
from jax.experimental import pallas as pl
from jax.experimental.pallas import tpu as pltpu
from jax.experimental.pallas import tpu_sc as plsc

N = 10000
E = 160000
IN_C = 256
HID = 512
NCLS = 128

NSUB = 16
NCORE = 2
LANE = 128
K = 128
N_PAD = 10240
ROWS_SUB = N_PAD // NSUB
DRAIN_PIECES = ROWS_SUB // 128
CHUNKS = 80
HCH = CHUNKS // 2
EPS_SUB = CHUNKS * K
E_PAD = EPS_SUB * NSUB
CH_TOT = E_PAD // K


def _make_segsum(fc):
    mesh = plsc.VectorSubcoreMesh(core_axis_name="c", subcore_axis_name="s")

    @functools.partial(
        pl.kernel,
        out_type=jax.ShapeDtypeStruct((fc * N_PAD, LANE), jnp.float32),
        mesh=mesh,
        scratch_types=[
            pltpu.VMEM((HCH, K), jnp.int32),
            pltpu.VMEM((HCH, K), jnp.int32),
            pltpu.VMEM((K, LANE), jnp.float32),
            pltpu.VMEM((K, LANE), jnp.float32),
            pltpu.VMEM_SHARED((N_PAD, LANE), jnp.float32),
            pltpu.SemaphoreType.DMA,
            pltpu.SemaphoreType.DMA,
        ],
    )
    def segsum(h_hbm, src_hbm, dst_hbm, out_hbm,
               sidx, didx, rows0, rows1, acc, sem0, sem1):
        c = lax.axis_index("c")
        s = lax.axis_index("s")

        def do_pass(p, carry):
            f = p * NCORE + c
            foff = f * N_PAD

            def zrow(r, carry2):
                for k in range(LANE // 16):
                    rows0[r, pl.ds(k * 16, 16)] = jnp.zeros((16,), jnp.float32)
                return carry2

            lax.fori_loop(0, 128, zrow, 0)

            def zpiece(k, carry2):
                pltpu.sync_copy(rows0, acc.at[pl.ds(s * ROWS_SUB + k * 128, 128)])
                return carry2

            lax.fori_loop(0, DRAIN_PIECES, zpiece, 0)
            plsc.subcore_barrier()

            def half(hf, carry2):
                sbase = f * CH_TOT + s * CHUNKS + hf * HCH
                dbase = s * CHUNKS + hf * HCH
                pltpu.sync_copy(src_hbm.at[pl.ds(sbase, HCH)], sidx)
                pltpu.sync_copy(dst_hbm.at[pl.ds(dbase, HCH)], didx)

                pltpu.async_copy(h_hbm.at[sidx.at[0]], rows0, sem0)

                def pair(g, carry3):
                    i0 = 2 * g
                    i1 = i0 + 1
                    pltpu.async_copy(h_hbm.at[sidx.at[i1]], rows1, sem1)
                    pltpu.make_async_copy(h_hbm.at[sidx.at[i0]], rows0, sem0).wait()
                    pltpu.sync_copy(rows0, acc.at[didx.at[i0]], add=True)

                    @pl.when(i0 + 2 < HCH)
                    def _():
                        pltpu.async_copy(h_hbm.at[sidx.at[i0 + 2]], rows0, sem0)

                    pltpu.make_async_copy(h_hbm.at[sidx.at[i1]], rows1, sem1).wait()
                    pltpu.sync_copy(rows1, acc.at[didx.at[i1]], add=True)
                    return carry3

                lax.fori_loop(0, HCH // 2, pair, 0)
                return carry2

            lax.fori_loop(0, 2, half, 0)
            plsc.subcore_barrier()

            def dpiece(k, carry2):
                r0 = s * ROWS_SUB + k * 128
                pltpu.sync_copy(acc.at[pl.ds(r0, 128)], rows0)
                pltpu.sync_copy(rows0, out_hbm.at[pl.ds(foff + r0, 128)])
                return carry2

            lax.fori_loop(0, DRAIN_PIECES, dpiece, 0)
            plsc.subcore_barrier()
            return carry

        lax.fori_loop(0, fc // NCORE, do_pass, 0)

    return segsum


_segsum2 = _make_segsum(2)
_segsum4 = _make_segsum(4)


def _make_mlp(fc_in, hid, fc_out, relu_out, bn=512):
    out_c = fc_out * LANE

    def body(scale_ref, agg_ref, h_ref, wa_ref, ba_ref, wb_ref, bb_ref, out_ref):
        scale = scale_ref[0]
        acc = jnp.zeros((bn, hid), jnp.float32)
        for f in range(fc_in):
            t = agg_ref[f] + scale * h_ref[f]
            acc = acc + jnp.dot(t, wa_ref[f], preferred_element_type=jnp.float32)
        h1 = jnp.maximum(acc + ba_ref[...], 0.0)
        h2 = jnp.dot(h1, wb_ref[...], preferred_element_type=jnp.float32) + bb_ref[...]
        if relu_out:
            h2 = jnp.maximum(h2, 0.0)
        for f2 in range(fc_out):
            out_ref[f2] = h2[:, f2 * LANE:(f2 + 1) * LANE]

    return pl.pallas_call(
        body,
        grid=(N_PAD // bn,),
        in_specs=[
            pl.BlockSpec(memory_space=pltpu.SMEM),
            pl.BlockSpec((fc_in, bn, LANE), lambda n: (0, n, 0)),
            pl.BlockSpec((fc_in, bn, LANE), lambda n: (0, n, 0)),
            pl.BlockSpec((fc_in, LANE, hid), lambda n: (0, 0, 0)),
            pl.BlockSpec((1, hid), lambda n: (0, 0)),
            pl.BlockSpec((hid, out_c), lambda n: (0, 0)),
            pl.BlockSpec((1, out_c), lambda n: (0, 0)),
        ],
        out_specs=pl.BlockSpec((fc_out, bn, LANE), lambda n: (0, n, 0)),
        out_shape=jax.ShapeDtypeStruct((fc_out, N_PAD, LANE), jnp.float32),
    )


_mlp0 = _make_mlp(IN_C // LANE, HID, HID // LANE, True)
_mlp1 = _make_mlp(HID // LANE, HID, HID // LANE, True)
_mlp2 = _make_mlp(HID // LANE, NCLS, NCLS // LANE, False)


def kernel(x, edge_index, W0a, b0a, W0b, b0b, W1a, b1a, W1b, b1b, W2a, b2a, W2b, b2b, eps):
    src = edge_index[0]
    dst = edge_index[1]
    pad = E_PAD - E
    srcp = jnp.concatenate([src, jnp.zeros((pad,), jnp.int32)]).reshape(CH_TOT, K)
    dstp = jnp.concatenate([dst, jnp.full((pad,), N_PAD - 1, jnp.int32)]).reshape(CH_TOT, K)
    offs2 = (jnp.arange(2, dtype=jnp.int32) * N_PAD).reshape(2, 1, 1)
    offs4 = (jnp.arange(4, dtype=jnp.int32) * N_PAD).reshape(4, 1, 1)
    srcp2 = (srcp[None] + offs2).reshape(2 * CH_TOT, K)
    srcp4 = (srcp[None] + offs4).reshape(4 * CH_TOT, K)

    fc0 = IN_C // LANE
    xp = jnp.pad(x, ((0, N_PAD - N), (0, 0)))
    x_flat = xp.reshape(N_PAD, fc0, LANE).transpose(1, 0, 2).reshape(fc0 * N_PAD, LANE)

    scales = (1.0 + eps).reshape(3, 1)

    agg0 = _segsum2(x_flat, srcp2, dstp)
    h1 = _mlp0(scales[0], agg0.reshape(fc0, N_PAD, LANE),
               x_flat.reshape(fc0, N_PAD, LANE),
               W0a.reshape(fc0, LANE, HID), b0a.reshape(1, HID),
               W0b, b0b.reshape(1, HID))

    fc1 = HID // LANE
    h1f = h1.reshape(fc1 * N_PAD, LANE)
    agg1 = _segsum4(h1f, srcp4, dstp)
    h2 = _mlp1(scales[1], agg1.reshape(fc1, N_PAD, LANE), h1,
               W1a.reshape(fc1, LANE, HID), b1a.reshape(1, HID),
               W1b, b1b.reshape(1, HID))

    h2f = h2.reshape(fc1 * N_PAD, LANE)
    agg2 = _segsum4(h2f, srcp4, dstp)
    out = _mlp2(scales[2], agg2.reshape(fc1, N_PAD, LANE), h2,
                W2a.reshape(fc1, LANE, NCLS), b2a.reshape(1, NCLS),
                W2b, b2b.reshape(1, NCLS))

    return out.reshape(N_PAD, NCLS)[:N]
